# baseline (speedup 1.0000x reference)
import functools

import jax
import jax.numpy as jnp
from jax import lax
from jax.experimental import pallas as pl
from jax.experimental.pallas import tpu as pltpu
from jax.experimental.pallas import tpu_sc as plsc

_L = 1600
_B = 128
_T = 16000
_M = 1664
_PAD_W = _T + 2 * _M
_NC = 2
_NS = 16
_NW = _NC * _NS
_ROWS_PER_W = _B // _NW
_LANES = 16
_CHUNKS = _T // _LANES
_TILE = 128
_NTILES = _T // _TILE


def _sc_shift_kernel(x_hbm, shifts_hbm, out_hbm, pad0, pad1, out0, out1,
                     out2, out3, sh_v, in_sems, out_sems):
  wid = lax.axis_index("s") * _NC + lax.axis_index("c")
  pads = [pad0, pad1]
  outs = [out0, out1, out2, out3]

  def in_pairs(j, c):
    r = wid * _ROWS_PER_W + j
    off = pl.multiple_of(c * _TILE, _TILE)
    return (x_hbm.at[r, pl.ds(off, _TILE)],
            pads[j % 2].at[0, pl.ds(_M + off, _TILE)], in_sems[j % 2])

  def out_pairs(j, c):
    r = wid * _ROWS_PER_W + j
    off = pl.multiple_of(c * _TILE, _TILE)
    return (outs[j].at[0, pl.ds(off, _TILE)],
            out_hbm.at[r, pl.ds(off, _TILE)], out_sems[j])

  def fire(pairs, j, lo=0, hi=_NTILES):
    @plsc.parallel_loop(lo, hi, 1, unroll=4)
    def body(c):
      pltpu.async_copy(*pairs(j, c))

  def drain_in(j):
    r = wid * _ROWS_PER_W + j
    pltpu.make_async_copy(x_hbm.at[r, pl.ds(0, _T)],
                          pads[j % 2].at[0, pl.ds(_M, _T)],
                          in_sems[j % 2]).wait()

  def drain_out(j):
    r = wid * _ROWS_PER_W + j
    pltpu.make_async_copy(outs[j].at[0, pl.ds(0, _T)],
                          out_hbm.at[r, pl.ds(0, _T)],
                          out_sems[j]).wait()

  fire(in_pairs, 0)
  fire(in_pairs, 1)

  pltpu.sync_copy(shifts_hbm, sh_v.at[pl.ds(0, _B)])
  zeros = jnp.zeros((_LANES,), jnp.float32)
  for b in range(2):
    for i in range(_M // _LANES):
      pads[b][0, pl.ds(i * _LANES, _LANES)] = zeros
      pads[b][0, pl.ds(_M + _T + i * _LANES, _LANES)] = zeros

  starts = [(_M + _L) - sh_v[pl.ds(wid * _ROWS_PER_W + j, _LANES)][0]
            for j in range(_ROWS_PER_W)]

  for j in range(_ROWS_PER_W):
    b = j % 2
    drain_in(j)
    start = starts[j]
    pad_b, out_b = pads[b], outs[j]

    for seg in range(5):
      k0, k1 = seg * 200, (seg + 1) * 200

      @plsc.parallel_loop(k0, k1, 1, unroll=16)
      def chunk(k):
        out_b[0, pl.ds(k * _LANES, _LANES)] = pad_b[0, pl.ds(
            start + k * _LANES, _LANES)]

      fire(out_pairs, j, seg * 25, (seg + 1) * 25)
    if j + 2 < _ROWS_PER_W:
      fire(in_pairs, j + 2)

  for j in range(_ROWS_PER_W):
    drain_out(j)


def kernel(x, shifts):
  mesh = plsc.VectorSubcoreMesh(core_axis_name="c", subcore_axis_name="s")
  f = functools.partial(
      pl.kernel,
      out_type=jax.ShapeDtypeStruct((_B, _T), jnp.float32),
      mesh=mesh,
      scratch_types=[
          pltpu.VMEM((1, _PAD_W), jnp.float32),
          pltpu.VMEM((1, _PAD_W), jnp.float32),
          pltpu.VMEM((1, _T), jnp.float32),
          pltpu.VMEM((1, _T), jnp.float32),
          pltpu.VMEM((1, _T), jnp.float32),
          pltpu.VMEM((1, _T), jnp.float32),
          pltpu.VMEM((_B + _LANES,), jnp.int32),
          [pltpu.SemaphoreType.DMA] * 2,
          [pltpu.SemaphoreType.DMA] * _ROWS_PER_W,
      ],
  )(_sc_shift_kernel)
  return f(x, shifts)


# speedup vs baseline: 1.0907x; 1.0907x over previous
import functools

import jax
import jax.numpy as jnp
from jax import lax
from jax.experimental import pallas as pl
from jax.experimental.pallas import tpu as pltpu
from jax.experimental.pallas import tpu_sc as plsc

_L = 1600
_B = 128
_T = 16000
_M = 1664
_PAD_W = _T + 2 * _M
_NC = 2
_NS = 16
_NW = _NC * _NS
_ROWS_PER_W = _B // _NW
_LANES = 16
_CHUNKS = _T // _LANES
_TILE = 128
_NTILES = _T // _TILE


def _sc_shift_kernel(x_hbm, shifts_hbm, out_hbm, pad0, pad1, out0, out1,
                     out2, out3, sh_v, in_sems, out_sems):
  wid = lax.axis_index("s") * _NC + lax.axis_index("c")
  pads = [pad0, pad1]
  outs = [out0, out1, out2, out3]

  def in_pairs(j, c):
    r = wid * _ROWS_PER_W + j
    off = pl.multiple_of(c * _TILE, _TILE)
    return (x_hbm.at[r, pl.ds(off, _TILE)],
            pads[j % 2].at[0, pl.ds(_M + off, _TILE)], in_sems[j % 2])

  def out_pairs(j, c):
    r = wid * _ROWS_PER_W + j
    off = pl.multiple_of(c * _TILE, _TILE)
    return (outs[j].at[0, pl.ds(off, _TILE)],
            out_hbm.at[r, pl.ds(off, _TILE)], out_sems[j])

  def fire(pairs, j, lo=0, hi=_NTILES):
    @plsc.parallel_loop(lo, hi, 1, unroll=4)
    def body(c):
      pltpu.async_copy(*pairs(j, c))

  def drain_in(j):
    r = wid * _ROWS_PER_W + j
    pltpu.make_async_copy(x_hbm.at[r, pl.ds(0, _T)],
                          pads[j % 2].at[0, pl.ds(_M, _T)],
                          in_sems[j % 2]).wait()

  def drain_out(j):
    r = wid * _ROWS_PER_W + j
    pltpu.make_async_copy(outs[j].at[0, pl.ds(0, _T)],
                          out_hbm.at[r, pl.ds(0, _T)],
                          out_sems[j]).wait()

  fire(in_pairs, 0)
  fire(in_pairs, 1)

  pltpu.sync_copy(shifts_hbm, sh_v.at[pl.ds(0, _B)])
  zeros = jnp.zeros((_LANES,), jnp.float32)
  for b in range(2):
    for i in range(_M // _LANES):
      pads[b][0, pl.ds(i * _LANES, _LANES)] = zeros
      pads[b][0, pl.ds(_M + _T + i * _LANES, _LANES)] = zeros

  starts = [(_M + _L) - sh_v[pl.ds(wid * _ROWS_PER_W + j, _LANES)][0]
            for j in range(_ROWS_PER_W)]

  for j in range(_ROWS_PER_W):
    b = j % 2
    drain_in(j)
    start = starts[j]
    pad_b, out_b = pads[b], outs[j]

    @plsc.parallel_loop(0, _CHUNKS, 1, unroll=16)
    def chunk(k):
      out_b[0, pl.ds(k * _LANES, _LANES)] = pad_b[0, pl.ds(start + k * _LANES,
                                                           _LANES)]

    fire(out_pairs, j)
    if j + 2 < _ROWS_PER_W:
      fire(in_pairs, j + 2)

  for j in range(_ROWS_PER_W):
    drain_out(j)


def kernel(x, shifts):
  mesh = plsc.VectorSubcoreMesh(core_axis_name="c", subcore_axis_name="s")
  f = functools.partial(
      pl.kernel,
      out_type=jax.ShapeDtypeStruct((_B, _T), jnp.float32),
      mesh=mesh,
      scratch_types=[
          pltpu.VMEM((1, _PAD_W), jnp.float32),
          pltpu.VMEM((1, _PAD_W), jnp.float32),
          pltpu.VMEM((1, _T), jnp.float32),
          pltpu.VMEM((1, _T), jnp.float32),
          pltpu.VMEM((1, _T), jnp.float32),
          pltpu.VMEM((1, _T), jnp.float32),
          pltpu.VMEM((_B + _LANES,), jnp.int32),
          [pltpu.SemaphoreType.DMA] * 2,
          [pltpu.SemaphoreType.DMA] * _ROWS_PER_W,
      ],
  )(_sc_shift_kernel)
  return f(x, shifts)


# speedup vs baseline: 1.1350x; 1.0406x over previous
import functools

import jax
import jax.numpy as jnp
from jax import lax
from jax.experimental import pallas as pl
from jax.experimental.pallas import tpu as pltpu
from jax.experimental.pallas import tpu_sc as plsc

_L = 1600
_B = 128
_T = 16000
_M = 1664
_PAD_W = _T + 2 * _M
_NC = 2
_NS = 16
_NW = _NC * _NS
_ROWS_PER_W = _B // _NW
_LANES = 16
_CHUNKS = _T // _LANES
_TILE = 128
_NTILES = _T // _TILE


def _sc_shift_kernel(x_hbm, shifts_hbm, out_hbm, pad0, pad1, out0, out1,
                     out2, out3, sh_v, in_sems, out_sems):
  wid = lax.axis_index("s") * _NC + lax.axis_index("c")
  pads = [pad0, pad1]
  outs = [out0, out1, out2, out3]

  def in_pairs(j, c):
    r = wid * _ROWS_PER_W + j
    off = pl.multiple_of(c * _TILE, _TILE)
    return (x_hbm.at[r, pl.ds(off, _TILE)],
            pads[j % 2].at[0, pl.ds(_M + off, _TILE)], in_sems[j % 2])

  def out_pairs(j, c):
    r = wid * _ROWS_PER_W + j
    off = pl.multiple_of(c * _TILE, _TILE)
    return (outs[j].at[0, pl.ds(off, _TILE)],
            out_hbm.at[r, pl.ds(off, _TILE)], out_sems[j])

  def fire(pairs, j, lo=0, hi=_NTILES):
    @plsc.parallel_loop(lo, hi, 1, unroll=4)
    def body(c):
      pltpu.async_copy(*pairs(j, c))

  def drain_in(j):
    r = wid * _ROWS_PER_W + j
    pltpu.make_async_copy(x_hbm.at[r, pl.ds(0, _T)],
                          pads[j % 2].at[0, pl.ds(_M, _T)],
                          in_sems[j % 2]).wait()

  def drain_out(j):
    r = wid * _ROWS_PER_W + j
    pltpu.make_async_copy(outs[j].at[0, pl.ds(0, _T)],
                          out_hbm.at[r, pl.ds(0, _T)],
                          out_sems[j]).wait()

  fire(in_pairs, 0)
  fire(in_pairs, 1)

  pltpu.sync_copy(shifts_hbm, sh_v.at[pl.ds(0, _B)])
  zeros = jnp.zeros((_LANES,), jnp.float32)
  for b in range(2):
    pad_z = pads[b]

    @plsc.parallel_loop(0, _M // _LANES, 1, unroll=4)
    def zero_margins(i):
      pad_z[0, pl.ds(i * _LANES, _LANES)] = zeros
      pad_z[0, pl.ds(_M + _T + i * _LANES, _LANES)] = zeros

  starts = [(_M + _L) - sh_v[pl.ds(wid * _ROWS_PER_W + j, _LANES)][0]
            for j in range(_ROWS_PER_W)]

  for j in range(_ROWS_PER_W):
    b = j % 2
    drain_in(j)
    start = starts[j]
    pad_b, out_b = pads[b], outs[j]

    @plsc.parallel_loop(0, _CHUNKS, 1, unroll=16)
    def chunk(k):
      out_b[0, pl.ds(k * _LANES, _LANES)] = pad_b[0, pl.ds(start + k * _LANES,
                                                           _LANES)]

    fire(out_pairs, j)
    if j + 2 < _ROWS_PER_W:
      fire(in_pairs, j + 2)

  for j in range(_ROWS_PER_W):
    drain_out(j)


def kernel(x, shifts):
  mesh = plsc.VectorSubcoreMesh(core_axis_name="c", subcore_axis_name="s")
  f = functools.partial(
      pl.kernel,
      out_type=jax.ShapeDtypeStruct((_B, _T), jnp.float32),
      mesh=mesh,
      scratch_types=[
          pltpu.VMEM((1, _PAD_W), jnp.float32),
          pltpu.VMEM((1, _PAD_W), jnp.float32),
          pltpu.VMEM((1, _T), jnp.float32),
          pltpu.VMEM((1, _T), jnp.float32),
          pltpu.VMEM((1, _T), jnp.float32),
          pltpu.VMEM((1, _T), jnp.float32),
          pltpu.VMEM((_B + _LANES,), jnp.int32),
          [pltpu.SemaphoreType.DMA] * 2,
          [pltpu.SemaphoreType.DMA] * _ROWS_PER_W,
      ],
  )(_sc_shift_kernel)
  return f(x, shifts)


# speedup vs baseline: 1.1363x; 1.0011x over previous
import functools

import jax
import jax.numpy as jnp
from jax import lax
from jax.experimental import pallas as pl
from jax.experimental.pallas import tpu as pltpu
from jax.experimental.pallas import tpu_sc as plsc

_L = 1600
_B = 128
_T = 16000
_M = 1664
_PAD_W = _T + 2 * _M
_NC = 2
_NS = 16
_NW = _NC * _NS
_ROWS_PER_W = _B // _NW
_LANES = 16
_CHUNKS = _T // _LANES
_TILE = 128
_NTILES = _T // _TILE


def _sc_shift_kernel(x_hbm, shifts_hbm, out_hbm, pad0, pad1, out0, out1,
                     out2, out3, sh_v, in_sems, out_sems):
  wid = lax.axis_index("s") * _NC + lax.axis_index("c")
  pads = [pad0, pad1]
  outs = [out0, out1, out2, out3]

  def in_pairs(j, c):
    r = wid * _ROWS_PER_W + j
    off = pl.multiple_of(c * _TILE, _TILE)
    return (x_hbm.at[r, pl.ds(off, _TILE)],
            pads[j % 2].at[0, pl.ds(_M + off, _TILE)], in_sems[j % 2])

  def out_pairs(j, c):
    r = wid * _ROWS_PER_W + j
    off = pl.multiple_of(c * _TILE, _TILE)
    return (outs[j].at[0, pl.ds(off, _TILE)],
            out_hbm.at[r, pl.ds(off, _TILE)], out_sems[j])

  def fire(pairs, j, lo=0, hi=_NTILES):
    @plsc.parallel_loop(lo, hi, 1, unroll=4)
    def body(c):
      pltpu.async_copy(*pairs(j, c))

  def drain_in(j):
    r = wid * _ROWS_PER_W + j
    pltpu.make_async_copy(x_hbm.at[r, pl.ds(0, _T)],
                          pads[j % 2].at[0, pl.ds(_M, _T)],
                          in_sems[j % 2]).wait()

  def drain_out(j):
    r = wid * _ROWS_PER_W + j
    pltpu.make_async_copy(outs[j].at[0, pl.ds(0, _T)],
                          out_hbm.at[r, pl.ds(0, _T)],
                          out_sems[j]).wait()

  fire(in_pairs, 0)
  fire(in_pairs, 1)

  pltpu.sync_copy(shifts_hbm, sh_v.at[pl.ds(0, _B)])
  zeros = jnp.zeros((_LANES,), jnp.float32)
  for b in range(2):
    pad_z = pads[b]

    @plsc.parallel_loop(0, _M // _LANES, 1, unroll=4)
    def zero_margins(i):
      pad_z[0, pl.ds(i * _LANES, _LANES)] = zeros
      pad_z[0, pl.ds(_M + _T + i * _LANES, _LANES)] = zeros

  starts = [(_M + _L) - sh_v[pl.ds(wid * _ROWS_PER_W + j, _LANES)][0]
            for j in range(_ROWS_PER_W)]

  for j in range(_ROWS_PER_W):
    b = j % 2
    drain_in(j)
    start = starts[j]
    pad_b, out_b = pads[b], outs[j]

    @plsc.parallel_loop(0, _CHUNKS, 1, unroll=8)
    def chunk(k):
      out_b[0, pl.ds(k * _LANES, _LANES)] = pad_b[0, pl.ds(start + k * _LANES,
                                                           _LANES)]

    fire(out_pairs, j)
    if j + 2 < _ROWS_PER_W:
      fire(in_pairs, j + 2)

  for j in range(_ROWS_PER_W):
    drain_out(j)


def kernel(x, shifts):
  mesh = plsc.VectorSubcoreMesh(core_axis_name="c", subcore_axis_name="s")
  f = functools.partial(
      pl.kernel,
      out_type=jax.ShapeDtypeStruct((_B, _T), jnp.float32),
      mesh=mesh,
      scratch_types=[
          pltpu.VMEM((1, _PAD_W), jnp.float32),
          pltpu.VMEM((1, _PAD_W), jnp.float32),
          pltpu.VMEM((1, _T), jnp.float32),
          pltpu.VMEM((1, _T), jnp.float32),
          pltpu.VMEM((1, _T), jnp.float32),
          pltpu.VMEM((1, _T), jnp.float32),
          pltpu.VMEM((_B + _LANES,), jnp.int32),
          [pltpu.SemaphoreType.DMA] * 2,
          [pltpu.SemaphoreType.DMA] * _ROWS_PER_W,
      ],
  )(_sc_shift_kernel)
  return f(x, shifts)
